# Initial kernel scaffold; baseline (speedup 1.0000x reference)
import jax
import jax.numpy as jnp
from jax.experimental import pallas as pl


def kernel(predict, target, weight):
    raise NotImplementedError("write your pallas kernel here")


# speedup vs baseline: 71.2393x; 71.2393x over previous
import functools

import jax
import jax.numpy as jnp
from jax import lax
from jax.experimental import pallas as pl
from jax.experimental.pallas import tpu as pltpu

B, C, H, W = 8, 2, 512, 512
N = B * H * W
MIN_KEPT = 100000
K_RANK = min(MIN_KEPT, N - 1)
LOGIT_T = 0.8472978603872037

GB, GH = 8, 8
ROWS = H // GH


def _dt(x_ref, t_ref):
    diff = x_ref[1] - x_ref[0]
    tt = t_ref[...]
    return jnp.where(tt == 1, diff, -diff), tt


def _loss(d, tt, w_ref):
    wt = jnp.where(tt == 1, w_ref[1], w_ref[0])
    return wt * (jnp.log1p(jnp.exp(-jnp.abs(d))) + jnp.maximum(-d, 0.0))


def _pass1_body(x_ref, t_ref, w_ref, cle_ref, clt_ref, sum_ref, acc_ref):
    b, h = pl.program_id(0), pl.program_id(1)
    first = jnp.logical_and(b == 0, h == 0)
    last = jnp.logical_and(b == GB - 1, h == GH - 1)

    @pl.when(first)
    def _():
        acc_ref[0] = 0
        acc_ref[1] = 0
        sum_ref[0, 0] = 0.0

    d, tt = _dt(x_ref, t_ref)
    L = jnp.float32(LOGIT_T)
    sel = d < L
    acc_ref[0] += jnp.sum((d <= L).astype(jnp.int32))
    acc_ref[1] += jnp.sum(sel.astype(jnp.int32))
    sum_ref[0, 0] += jnp.sum(jnp.where(sel, _loss(d, tt, w_ref), 0.0))

    @pl.when(last)
    def _():
        cle_ref[0, 0] = acc_ref[0]
        clt_ref[0, 0] = acc_ref[1]


def _key(d):
    bits = lax.bitcast_convert_type(d, jnp.int32)
    return jnp.where(bits >= 0, bits, bits ^ jnp.int32(0x7FFFFFFF))


def _bisect_body(x_ref, t_ref, w_ref, cnt_ref, sum_ref, sm_ref, sf_ref):
    j, b, h = pl.program_id(0), pl.program_id(1), pl.program_id(2)
    first = jnp.logical_and(b == 0, h == 0)

    @pl.when(jnp.logical_and(first, j == 0))
    def _():
        sm_ref[0] = jnp.int32(-2147483648)
        sm_ref[1] = 0
        sm_ref[2] = 0
        sf_ref[0] = 0.0

    @pl.when(jnp.logical_and(first, jnp.logical_and(j > 0, j <= 32)))
    def _():
        bump = jnp.where(
            sm_ref[1] <= K_RANK,
            lax.shift_left(jnp.int32(1), jnp.clip(32 - j, 0, 31)), 0)
        sm_ref[0] += bump
        sm_ref[1] = 0

    d, tt = _dt(x_ref, t_ref)
    key = _key(d)

    @pl.when(j < 32)
    def _():
        t = sm_ref[0] + lax.shift_left(jnp.int32(1), jnp.clip(31 - j, 0, 31))
        sm_ref[1] += jnp.sum((key < t).astype(jnp.int32))

    @pl.when(j == 32)
    def _():
        sel = key < sm_ref[0]
        sm_ref[2] += jnp.sum(sel.astype(jnp.int32))
        sf_ref[0] += jnp.sum(jnp.where(sel, _loss(d, tt, w_ref), 0.0))

    @pl.when(j == 33)
    def _():
        cnt_ref[0, 0] = sm_ref[2]
        sum_ref[0, 0] = sf_ref[0]


def _mk_specs(three_d):
    off = 1 if three_d else 0

    def xmap(*ids):
        return (ids[off], ids[off + 1], 0)

    def tmap(*ids):
        return (ids[off] * GH + ids[off + 1], 0)

    return [
        pl.BlockSpec((2, ROWS, W), xmap),
        pl.BlockSpec((ROWS, W), tmap),
        pl.BlockSpec(memory_space=pltpu.SMEM),
    ]


def _scalar_outs(dtypes):
    return (
        tuple(jax.ShapeDtypeStruct((1, 1), dt) for dt in dtypes),
        tuple(pl.BlockSpec(memory_space=pltpu.SMEM) for _ in dtypes),
    )


@jax.jit
def kernel(predict, target, weight):
    xv = predict.reshape(B * C, H, W)
    tv = target.astype(jnp.int32).reshape(B * H, W)
    wv = weight.astype(jnp.float32)

    out_shape, out_specs = _scalar_outs((jnp.int32, jnp.int32, jnp.float32))
    cle, clt, s_lt = pl.pallas_call(
        _pass1_body,
        grid=(GB, GH),
        in_specs=_mk_specs(False),
        out_specs=list(out_specs),
        out_shape=list(out_shape),
        scratch_shapes=[pltpu.SMEM((2,), jnp.int32)],
    )(xv, tv, wv)

    cle = cle[0, 0]
    clt = clt[0, 0]
    s_lt = s_lt[0, 0]

    def common(_):
        return s_lt, clt

    def rare(_):
        o_shape, o_specs = _scalar_outs((jnp.int32, jnp.float32))
        cnt, tot = pl.pallas_call(
            _bisect_body,
            grid=(34, GB, GH),
            in_specs=_mk_specs(True),
            out_specs=list(o_specs),
            out_shape=list(o_shape),
            scratch_shapes=[pltpu.SMEM((3,), jnp.int32),
                            pltpu.SMEM((1,), jnp.float32)],
        )(xv, tv, wv)
        return tot[0, 0], cnt[0, 0]

    total, cnt = lax.cond(cle >= K_RANK + 1, common, rare, operand=None)
    return jnp.where(cnt == 0, total,
                     total / jnp.maximum(cnt, 1).astype(jnp.float32))


# speedup vs baseline: 115.5783x; 1.6224x over previous
import functools

import jax
import jax.numpy as jnp
from jax import lax
from jax.experimental import pallas as pl
from jax.experimental.pallas import tpu as pltpu

B, C, H, W = 8, 2, 512, 512
N = B * H * W
MIN_KEPT = 100000
K_RANK = min(MIN_KEPT, N - 1)
LOGIT_T = 0.8472978603872037

GB, GH = 8, 4
ROWS = H // GH


def _dt(x_ref, t_ref):
    diff = x_ref[1] - x_ref[0]
    tt = t_ref[...]
    return jnp.where(tt == 1, diff, -diff), tt


def _loss(d, tt, w_ref):
    wt = jnp.where(tt == 1, w_ref[1], w_ref[0])
    return wt * (jnp.log1p(jnp.exp(-jnp.abs(d))) + jnp.maximum(-d, 0.0))


def _pass1_body(x_ref, t_ref, w_ref, cle_ref, clt_ref, sum_ref,
                a_le, a_lt, a_sum):
    b, h = pl.program_id(0), pl.program_id(1)
    first = jnp.logical_and(b == 0, h == 0)
    last = jnp.logical_and(b == GB - 1, h == GH - 1)

    @pl.when(first)
    def _():
        a_le[...] = jnp.zeros_like(a_le)
        a_lt[...] = jnp.zeros_like(a_lt)
        a_sum[...] = jnp.zeros_like(a_sum)

    d, tt = _dt(x_ref, t_ref)
    L = jnp.float32(LOGIT_T)
    sel = d < L
    one = jnp.float32(1.0)
    zero = jnp.float32(0.0)
    a_le[...] += jnp.where(d <= L, one, zero)
    a_lt[...] += jnp.where(sel, one, zero)
    a_sum[...] += jnp.where(sel, _loss(d, tt, w_ref), zero)

    @pl.when(last)
    def _():
        cle_ref[0, 0] = jnp.sum(a_le[...]).astype(jnp.int32)
        clt_ref[0, 0] = jnp.sum(a_lt[...]).astype(jnp.int32)
        sum_ref[0, 0] = jnp.sum(a_sum[...])


def _key(d):
    bits = lax.bitcast_convert_type(d, jnp.int32)
    return jnp.where(bits >= 0, bits, bits ^ jnp.int32(0x7FFFFFFF))


def _bisect_body(x_ref, t_ref, w_ref, cnt_ref, sum_ref, sm_ref, sf_ref):
    j, b, h = pl.program_id(0), pl.program_id(1), pl.program_id(2)
    first = jnp.logical_and(b == 0, h == 0)

    @pl.when(jnp.logical_and(first, j == 0))
    def _():
        sm_ref[0] = jnp.int32(-2147483648)
        sm_ref[1] = 0
        sm_ref[2] = 0
        sf_ref[0] = 0.0

    @pl.when(jnp.logical_and(first, jnp.logical_and(j > 0, j <= 32)))
    def _():
        bump = jnp.where(
            sm_ref[1] <= K_RANK,
            lax.shift_left(jnp.int32(1), jnp.clip(32 - j, 0, 31)), 0)
        sm_ref[0] += bump
        sm_ref[1] = 0

    d, tt = _dt(x_ref, t_ref)
    key = _key(d)

    @pl.when(j < 32)
    def _():
        t = sm_ref[0] + lax.shift_left(jnp.int32(1), jnp.clip(31 - j, 0, 31))
        sm_ref[1] += jnp.sum((key < t).astype(jnp.int32))

    @pl.when(j == 32)
    def _():
        sel = key < sm_ref[0]
        sm_ref[2] += jnp.sum(sel.astype(jnp.int32))
        sf_ref[0] += jnp.sum(jnp.where(sel, _loss(d, tt, w_ref), 0.0))

    @pl.when(j == 33)
    def _():
        cnt_ref[0, 0] = sm_ref[2]
        sum_ref[0, 0] = sf_ref[0]


def _mk_specs(three_d):
    off = 1 if three_d else 0

    def xmap(*ids):
        return (ids[off], ids[off + 1], 0)

    def tmap(*ids):
        return (ids[off] * GH + ids[off + 1], 0)

    return [
        pl.BlockSpec((2, ROWS, W), xmap),
        pl.BlockSpec((ROWS, W), tmap),
        pl.BlockSpec(memory_space=pltpu.SMEM),
    ]


def _scalar_outs(dtypes):
    return (
        tuple(jax.ShapeDtypeStruct((1, 1), dt) for dt in dtypes),
        tuple(pl.BlockSpec(memory_space=pltpu.SMEM) for _ in dtypes),
    )


@jax.jit
def kernel(predict, target, weight):
    xv = predict.reshape(B * C, H, W)
    tv = target.astype(jnp.int32).reshape(B * H, W)
    wv = weight.astype(jnp.float32)

    out_shape, out_specs = _scalar_outs((jnp.int32, jnp.int32, jnp.float32))
    cle, clt, s_lt = pl.pallas_call(
        _pass1_body,
        grid=(GB, GH),
        in_specs=_mk_specs(False),
        out_specs=list(out_specs),
        out_shape=list(out_shape),
        scratch_shapes=[pltpu.VMEM((ROWS, W), jnp.float32)] * 3,
    )(xv, tv, wv)

    cle = cle[0, 0]
    clt = clt[0, 0]
    s_lt = s_lt[0, 0]

    def common(_):
        return s_lt, clt

    def rare(_):
        o_shape, o_specs = _scalar_outs((jnp.int32, jnp.float32))
        cnt, tot = pl.pallas_call(
            _bisect_body,
            grid=(34, GB, GH),
            in_specs=_mk_specs(True),
            out_specs=list(o_specs),
            out_shape=list(o_shape),
            scratch_shapes=[pltpu.SMEM((3,), jnp.int32),
                            pltpu.SMEM((1,), jnp.float32)],
        )(xv, tv, wv)
        return tot[0, 0], cnt[0, 0]

    total, cnt = lax.cond(cle >= K_RANK + 1, common, rare, operand=None)
    return jnp.where(cnt == 0, total,
                     total / jnp.maximum(cnt, 1).astype(jnp.float32))
